# baseline (speedup 1.0000x reference)
import jax
import jax.numpy as jnp
from jax import lax
from jax.experimental import pallas as pl
from jax.experimental.pallas import tpu as pltpu
from jax.experimental.pallas import tpu_sc as plsc

N = 4096
L = 16
NSUB = 32
CHUNK = N // NSUB
UNROLL = 4
NEG_BIG = -1e30


def _sc_body(p_hbm, lab_hbm, sums_hbm, meta_hbm,
             pv, labv, qbuf, accbuf, obuf, mbuf):
    c = lax.axis_index("c")
    s = lax.axis_index("s")
    wid = c * 16 + s
    base = wid * CHUNK

    pltpu.sync_copy(p_hbm, pv.at[pl.ds(0, N)])
    pltpu.sync_copy(lab_hbm, labv.at[pl.ds(0, N)])

    neg = jnp.full((L,), NEG_BIG, dtype=jnp.float32)

    def build(jv, carry):
        v = pv[pl.ds(jv * L, L)]
        m = labv[pl.ds(jv * L, L)] == 0
        qbuf[pl.ds(jv * L, L)] = jnp.where(m, v, neg)
        return carry

    lax.fori_loop(0, N // L, build, jnp.int32(0))

    zero = jnp.zeros((L,), dtype=jnp.float32)
    n_steps = N // (UNROLL * L)

    def clear(k, carry):
        accbuf[pl.ds(k * L, L)] = zero
        return carry

    lax.fori_loop(0, UNROLL, clear, jnp.int32(0))

    def outer(i, n1w):
        li = labv[pl.ds(base + i, L)][0]
        pi = pv[pl.ds(base + i, L)][0]

        @pl.when(li == 1)
        def _():
            def inner(k, accs):
                b0, b1, b2, b3 = accs
                q0 = qbuf[pl.ds(k * (UNROLL * L), L)]
                q1 = qbuf[pl.ds(k * (UNROLL * L) + L, L)]
                q2 = qbuf[pl.ds(k * (UNROLL * L) + 2 * L, L)]
                q3 = qbuf[pl.ds(k * (UNROLL * L) + 3 * L, L)]
                b0 = b0 + jnp.maximum(q0 - pi, 0.0)
                b1 = b1 + jnp.maximum(q1 - pi, 0.0)
                b2 = b2 + jnp.maximum(q2 - pi, 0.0)
                b3 = b3 + jnp.maximum(q3 - pi, 0.0)
                return (b0, b1, b2, b3)

            accs = (accbuf[pl.ds(0, L)], accbuf[pl.ds(L, L)],
                    accbuf[pl.ds(2 * L, L)], accbuf[pl.ds(3 * L, L)])
            b0, b1, b2, b3 = lax.fori_loop(0, n_steps, inner, accs)
            accbuf[pl.ds(0, L)] = b0
            accbuf[pl.ds(L, L)] = b1
            accbuf[pl.ds(2 * L, L)] = b2
            accbuf[pl.ds(3 * L, L)] = b3

        return n1w + jnp.where(li == 1, jnp.int32(1), jnp.int32(0))

    n1w = lax.fori_loop(0, CHUNK, outer, jnp.int32(0))

    obuf[...] = ((accbuf[pl.ds(0, L)] + accbuf[pl.ds(L, L)])
                 + (accbuf[pl.ds(2 * L, L)] + accbuf[pl.ds(3 * L, L)]))
    pltpu.sync_copy(obuf, sums_hbm.at[wid])

    lane = lax.broadcasted_iota(jnp.int32, (L,), 0)
    n0w = jnp.int32(CHUNK) - n1w
    meta = jnp.where(lane == 0, n1w.astype(jnp.float32),
                     jnp.where(lane == 1, n0w.astype(jnp.float32), 0.0))
    mbuf[...] = meta
    pltpu.sync_copy(mbuf, meta_hbm.at[wid])


_mesh = plsc.VectorSubcoreMesh(core_axis_name="c", subcore_axis_name="s")

_pairwise_sc = pl.kernel(
    _sc_body,
    out_type=(jax.ShapeDtypeStruct((NSUB, L), jnp.float32),
              jax.ShapeDtypeStruct((NSUB, L), jnp.float32)),
    mesh=_mesh,
    scratch_types=[
        pltpu.VMEM((N + L,), jnp.float32),
        pltpu.VMEM((N + L,), jnp.int32),
        pltpu.VMEM((N,), jnp.float32),
        pltpu.VMEM((UNROLL * L,), jnp.float32),
        pltpu.VMEM((L,), jnp.float32),
        pltpu.VMEM((L,), jnp.float32),
    ],
)


@jax.jit
def kernel(prediction, label):
    p = prediction.reshape(-1)
    lab = label.reshape(-1)
    sums, meta = _pairwise_sc(p, lab)
    loss_sum = jnp.sum(sums)
    n1 = jnp.sum(meta[:, 0])
    n0 = jnp.sum(meta[:, 1])
    count = n1 * n0
    return jnp.where(count > 0, loss_sum / count, jnp.float32(0.0))


# speedup vs baseline: 1.1130x; 1.1130x over previous
import jax
import jax.numpy as jnp
from jax import lax
from jax.experimental import pallas as pl
from jax.experimental.pallas import tpu as pltpu
from jax.experimental.pallas import tpu_sc as plsc

N = 4096
L = 16
NSUB = 32
CHUNK = N // NSUB
QV = 4
NBLK = N // (QV * L)
NEG_BIG = -1e30


def _sc_body(p_hbm, lab_hbm, sums_hbm, meta_hbm,
             pv, labv, qbuf, rsm, obuf, mbuf):
    c = lax.axis_index("c")
    s = lax.axis_index("s")
    wid = c * 16 + s
    base = wid * CHUNK

    pltpu.sync_copy(p_hbm, pv.at[pl.ds(0, N)])
    pltpu.sync_copy(lab_hbm, labv.at[pl.ds(0, N)])

    neg = jnp.full((L,), NEG_BIG, dtype=jnp.float32)

    def build(jv, carry):
        v = pv[pl.ds(jv * L, L)]
        m = labv[pl.ds(jv * L, L)] == 0
        qbuf[pl.ds(jv * L, L)] = jnp.where(m, v, neg)
        return carry

    lax.fori_loop(0, N // L, build, jnp.int32(0), unroll=4)

    def rowc(g, cnt):
        lv = labv[pl.ds(base + g * L, L)]
        pvv = pv[pl.ds(base + g * L, L)]
        for k in range(L):
            li = lv[k]
            pi = pvv[k]

            @pl.when(li == 1)
            def _(cnt=cnt, pi=pi):
                rsm[cnt] = pi

            cnt = cnt + jnp.where(li == 1, jnp.int32(1), jnp.int32(0))
        return cnt

    n1w = lax.fori_loop(0, CHUNK // L, rowc, jnp.int32(0))

    rsm[n1w] = jnp.float32(-NEG_BIG)
    rsm[n1w + 1] = jnp.float32(-NEG_BIG)
    n_pair = (n1w + 1) // 2

    zero = jnp.zeros((L,), dtype=jnp.float32)

    def qblock(b, gaccs):
        q0 = qbuf[pl.ds(b * (QV * L), L)]
        q1 = qbuf[pl.ds(b * (QV * L) + L, L)]
        q2 = qbuf[pl.ds(b * (QV * L) + 2 * L, L)]
        q3 = qbuf[pl.ds(b * (QV * L) + 3 * L, L)]

        def rows(i, accs):
            b0, b1, b2, b3 = accs
            pi = rsm[2 * i]
            pj = rsm[2 * i + 1]
            b0 = b0 + jnp.maximum(q0 - pi, 0.0)
            b1 = b1 + jnp.maximum(q1 - pi, 0.0)
            b2 = b2 + jnp.maximum(q2 - pi, 0.0)
            b3 = b3 + jnp.maximum(q3 - pi, 0.0)
            b0 = b0 + jnp.maximum(q0 - pj, 0.0)
            b1 = b1 + jnp.maximum(q1 - pj, 0.0)
            b2 = b2 + jnp.maximum(q2 - pj, 0.0)
            b3 = b3 + jnp.maximum(q3 - pj, 0.0)
            return (b0, b1, b2, b3)

        return lax.fori_loop(0, n_pair, rows, gaccs)

    a0, a1, a2, a3 = lax.fori_loop(
        0, NBLK, qblock, (zero, zero, zero, zero))

    obuf[...] = (a0 + a1) + (a2 + a3)
    pltpu.sync_copy(obuf, sums_hbm.at[wid])

    lane = lax.broadcasted_iota(jnp.int32, (L,), 0)
    n0w = jnp.int32(CHUNK) - n1w
    meta = jnp.where(lane == 0, n1w.astype(jnp.float32),
                     jnp.where(lane == 1, n0w.astype(jnp.float32), 0.0))
    mbuf[...] = meta
    pltpu.sync_copy(mbuf, meta_hbm.at[wid])


_mesh = plsc.VectorSubcoreMesh(core_axis_name="c", subcore_axis_name="s")

_pairwise_sc = pl.kernel(
    _sc_body,
    out_type=(jax.ShapeDtypeStruct((NSUB, L), jnp.float32),
              jax.ShapeDtypeStruct((NSUB, L), jnp.float32)),
    mesh=_mesh,
    scratch_types=[
        pltpu.VMEM((N + L,), jnp.float32),
        pltpu.VMEM((N + L,), jnp.int32),
        pltpu.VMEM((N,), jnp.float32),
        pltpu.SMEM((CHUNK + 2,), jnp.float32),
        pltpu.VMEM((L,), jnp.float32),
        pltpu.VMEM((L,), jnp.float32),
    ],
)


@jax.jit
def kernel(prediction, label):
    p = prediction.reshape(-1)
    lab = label.reshape(-1)
    sums, meta = _pairwise_sc(p, lab)
    loss_sum = jnp.sum(sums)
    n1 = jnp.sum(meta[:, 0])
    n0 = jnp.sum(meta[:, 1])
    count = n1 * n0
    return jnp.where(count > 0, loss_sum / count, jnp.float32(0.0))


# speedup vs baseline: 1.2184x; 1.0947x over previous
import jax
import jax.numpy as jnp
from jax import lax
from jax.experimental import pallas as pl
from jax.experimental.pallas import tpu as pltpu
from jax.experimental.pallas import tpu_sc as plsc

N = 4096
L = 16
NSUB = 32
CHUNK = N // NSUB
QV = 4
NBLK = N // (QV * L)
NEG_BIG = -1e30


def _sc_body(p_hbm, lab_hbm, sums_hbm, meta_hbm,
             pv, labv, qbuf, rsm, obuf, mbuf):
    c = lax.axis_index("c")
    s = lax.axis_index("s")
    wid = c * 16 + s
    base = wid * CHUNK

    pltpu.sync_copy(p_hbm, pv.at[pl.ds(0, N)])
    pltpu.sync_copy(lab_hbm, labv.at[pl.ds(0, N)])

    neg = jnp.full((L,), NEG_BIG, dtype=jnp.float32)

    def build(jv, carry):
        v = pv[pl.ds(jv * L, L)]
        m = labv[pl.ds(jv * L, L)] == 0
        qbuf[pl.ds(jv * L, L)] = jnp.where(m, v, neg)
        return carry

    lax.fori_loop(0, N // L, build, jnp.int32(0), unroll=4)

    def rowc(g, cnt):
        lv = labv[pl.ds(base + g * L, L)]
        pvv = pv[pl.ds(base + g * L, L)]
        for k in range(L):
            li = lv[k]
            pi = pvv[k]

            @pl.when(li == 1)
            def _(cnt=cnt, pi=pi):
                rsm[cnt] = pi

            cnt = cnt + jnp.where(li == 1, jnp.int32(1), jnp.int32(0))
        return cnt

    n1w = lax.fori_loop(0, CHUNK // L, rowc, jnp.int32(0))

    rsm[n1w] = jnp.float32(-NEG_BIG)
    rsm[n1w + 1] = jnp.float32(-NEG_BIG)
    n_pair = (n1w + 1) // 2

    zero = jnp.zeros((L,), dtype=jnp.float32)

    def qblock(b, gaccs):
        q0 = qbuf[pl.ds(b * (QV * L), L)]
        q1 = qbuf[pl.ds(b * (QV * L) + L, L)]
        q2 = qbuf[pl.ds(b * (QV * L) + 2 * L, L)]
        q3 = qbuf[pl.ds(b * (QV * L) + 3 * L, L)]

        def rows(i, accs):
            b0, b1, b2, b3 = accs
            pi = rsm[2 * i]
            pj = rsm[2 * i + 1]
            b0 = b0 + jnp.maximum(q0 - pi, 0.0)
            b1 = b1 + jnp.maximum(q1 - pi, 0.0)
            b2 = b2 + jnp.maximum(q2 - pi, 0.0)
            b3 = b3 + jnp.maximum(q3 - pi, 0.0)
            b0 = b0 + jnp.maximum(q0 - pj, 0.0)
            b1 = b1 + jnp.maximum(q1 - pj, 0.0)
            b2 = b2 + jnp.maximum(q2 - pj, 0.0)
            b3 = b3 + jnp.maximum(q3 - pj, 0.0)
            return (b0, b1, b2, b3)

        return lax.fori_loop(0, n_pair, rows, gaccs)

    a0, a1, a2, a3 = lax.fori_loop(
        0, NBLK, qblock, (zero, zero, zero, zero))

    obuf[...] = (a0 + a1) + (a2 + a3)
    pltpu.sync_copy(obuf, sums_hbm.at[wid])

    lane = lax.broadcasted_iota(jnp.int32, (L,), 0)
    n0w = jnp.int32(CHUNK) - n1w
    meta = jnp.where(lane == 0, n1w.astype(jnp.float32),
                     jnp.where(lane == 1, n0w.astype(jnp.float32), 0.0))
    mbuf[...] = meta
    pltpu.sync_copy(mbuf, meta_hbm.at[wid])


_mesh = plsc.VectorSubcoreMesh(core_axis_name="c", subcore_axis_name="s")

_pairwise_sc = pl.kernel(
    _sc_body,
    out_type=(jax.ShapeDtypeStruct((NSUB, L), jnp.float32),
              jax.ShapeDtypeStruct((NSUB, L), jnp.float32)),
    mesh=_mesh,
    scratch_types=[
        pltpu.VMEM((N + L,), jnp.float32),
        pltpu.VMEM((N + L,), jnp.int32),
        pltpu.VMEM((N,), jnp.float32),
        pltpu.SMEM((CHUNK + 2,), jnp.float32),
        pltpu.VMEM((L,), jnp.float32),
        pltpu.VMEM((L,), jnp.float32),
    ],
)


ROWS_PER_TILE = 128
NTILES = N // ROWS_PER_TILE


def _tc_body(pc_ref, lc_ref, pr_ref, lr_ref, sum_ref, cnt_ref):
    r = jnp.where(lc_ref[...] == 1, pc_ref[...], -NEG_BIG)
    q = jnp.where(lr_ref[...] == 0, pr_ref[...], NEG_BIG)
    contrib = jnp.maximum(q - r, 0.0)
    sum_ref[...] = jnp.sum(contrib).reshape(1, 1, 1)
    cnt_ref[...] = jnp.sum(lc_ref[...].astype(jnp.float32)).reshape(1, 1, 1)


_pairwise_tc = pl.pallas_call(
    _tc_body,
    grid=(NTILES,),
    in_specs=[
        pl.BlockSpec((ROWS_PER_TILE, 1), lambda i: (i, 0)),
        pl.BlockSpec((ROWS_PER_TILE, 1), lambda i: (i, 0)),
        pl.BlockSpec((1, N), lambda i: (0, 0)),
        pl.BlockSpec((1, N), lambda i: (0, 0)),
    ],
    out_specs=[
        pl.BlockSpec((1, 1, 1), lambda i: (i, 0, 0)),
        pl.BlockSpec((1, 1, 1), lambda i: (i, 0, 0)),
    ],
    out_shape=[
        jax.ShapeDtypeStruct((NTILES, 1, 1), jnp.float32),
        jax.ShapeDtypeStruct((NTILES, 1, 1), jnp.float32),
    ],
)


@jax.jit
def kernel(prediction, label):
    pcol = prediction.reshape(N, 1)
    lcol = label.reshape(N, 1)
    prow = prediction.reshape(1, N)
    lrow = label.reshape(1, N)
    sums, n1t = _pairwise_tc(pcol, lcol, prow, lrow)
    loss_sum = jnp.sum(sums)
    n1 = jnp.sum(n1t)
    n0 = jnp.float32(N) - n1
    count = n1 * n0
    return jnp.where(count > 0, loss_sum / count, jnp.float32(0.0))
